# baseline (speedup 1.0000x reference)
import functools

import jax
import jax.numpy as jnp
from jax import lax
from jax.experimental import pallas as pl
from jax.experimental.pallas import tpu as pltpu
from jax.experimental.pallas import tpu_sc as plsc

_B = 16384
_D = 128
_ND = 32
_NC = 2
_NS = 16
_NW = _NC * _NS
_RPW = _B // _NW
_NCHUNK = 4
_RPC = _RPW // _NCHUNK

_ROTS = ((13, 15, 26, 6), (17, 29, 16, 24))
_KS = (0x0, 0x1, 0x1BD11BDB)


def _rotl(x, r):
    return lax.shift_left(x, jnp.uint32(r)) | lax.shift_right_logical(
        x, jnp.uint32(32 - r))


def _threefry_noise(c2):
    x1 = c2 + jnp.uint32(_KS[1])
    x0 = x1
    x1 = x0 ^ _rotl(x1, 13)
    for r in (15, 26, 6):
        x0 = x0 + x1
        x1 = x0 ^ _rotl(x1, r)
    x0 = x0 + jnp.uint32(_KS[1])
    x1 = x1 + jnp.uint32(_KS[2] + 1)
    for g in range(1, 5):
        for r in _ROTS[g % 2]:
            x0 = x0 + x1
            x1 = x0 ^ _rotl(x1, r)
        a = _KS[(g + 1) % 3]
        b = (_KS[(g + 2) % 3] + g + 1) & 0xFFFFFFFF
        if a:
            x0 = x0 + jnp.uint32(a)
        x1 = x1 + jnp.uint32(b)
    bits = x0 ^ x1
    mant = lax.convert_element_type(
        lax.shift_right_logical(bits, jnp.uint32(9)), jnp.int32)
    return lax.convert_element_type(mant, jnp.float32) * jnp.float32(
        1.0 / 8388608.0) - jnp.float32(0.5)


def _sc_body(in_hbm, shift_hbm, scale_hbm, idx_hbm, out_hbm,
             buf, shift_v, scale_v, idx_v,
             sem_p, sem_o, sem_i0, sem_i1, sem_i2, sem_i3):
    c = lax.axis_index("c")
    s = lax.axis_index("s")
    wid = s * _NC + c
    row0 = wid * _RPW

    p0 = pltpu.async_copy(shift_hbm, shift_v, sem_p)
    p1 = pltpu.async_copy(scale_hbm, scale_v, sem_p)
    p2 = pltpu.async_copy(idx_hbm, idx_v, sem_p)
    sems_in = (sem_i0, sem_i1, sem_i2, sem_i3)
    copies_in = []
    for k in range(_NCHUNK):
        sl = pl.ds(k * _RPC, _RPC)
        hsl = pl.ds(row0 + k * _RPC, _RPC)
        copies_in.append(
            pltpu.async_copy(in_hbm.at[hsl], buf.at[sl], sems_in[k]))
    p0.wait()
    p1.wait()
    p2.wait()

    lane_u = lax.iota(jnp.uint32, 16)
    lane_i = lax.iota(jnp.int32, 16)
    cols = [idx_v[pl.ds(0, 16)], idx_v[pl.ds(16, 16)]]
    shs = [shift_v[pl.ds(0, 16)], shift_v[pl.ds(16, 16)]]
    scs = [scale_v[pl.ds(0, 16)], scale_v[pl.ds(16, 16)]]
    one = jnp.float32(1.0)
    invs = [one / scs[0], one / scs[1]]
    base0 = lax.convert_element_type(row0 * _ND, jnp.uint32)

    def do_row(lr):
        row_vec = lane_i * 0 + lr
        base = base0 + lax.convert_element_type(lr * _ND, jnp.uint32)
        for h in range(2):
            c2 = lane_u + (base + jnp.uint32(16 * h))
            n = _threefry_noise(c2)
            x = plsc.load_gather(buf, [row_vec, cols[h]])
            d = x * scs[h] + shs[h]
            new = (d + n - shs[h]) * invs[h]
            plsc.store_scatter(buf, [row_vec, cols[h]], new)

    copies_out = []
    for k in range(_NCHUNK):
        copies_in[k].wait()

        def chunk_step(t, carry, k=k):
            lr = k * _RPC + t * 2
            do_row(lr)
            do_row(lr + 1)
            return carry

        lax.fori_loop(0, _RPC // 2, chunk_step, 0)
        sl = pl.ds(k * _RPC, _RPC)
        hsl = pl.ds(row0 + k * _RPC, _RPC)
        copies_out.append(
            pltpu.async_copy(buf.at[sl], out_hbm.at[hsl], sem_o))
    for cp in copies_out:
        cp.wait()


@functools.lru_cache(maxsize=1)
def _sc_call():
    return pl.kernel(
        _sc_body,
        out_type=jax.ShapeDtypeStruct((_B, _D), jnp.float32),
        mesh=plsc.VectorSubcoreMesh(core_axis_name="c", subcore_axis_name="s",
                                    num_cores=_NC, num_subcores=_NS),
        compiler_params=pltpu.CompilerParams(needs_layout_passes=False),
        scratch_types=[
            pltpu.VMEM((_RPW, _D), jnp.float32),
            pltpu.VMEM((_ND,), jnp.float32),
            pltpu.VMEM((_ND,), jnp.float32),
            pltpu.VMEM((_ND,), jnp.int32),
            pltpu.SemaphoreType.DMA,
            pltpu.SemaphoreType.DMA,
            pltpu.SemaphoreType.DMA,
            pltpu.SemaphoreType.DMA,
            pltpu.SemaphoreType.DMA,
            pltpu.SemaphoreType.DMA,
        ],
    )


def kernel(inputs, discrete_shift, discrete_scale, discrete_idx):
    return _sc_call()(inputs, discrete_shift, discrete_scale, discrete_idx)
